# baseline (speedup 1.0000x reference)
import jax
import jax.numpy as jnp
from jax import lax
from jax.experimental import pallas as pl
from jax.experimental.pallas import tpu as pltpu
from jax.experimental.pallas import tpu_sc as plsc

N = 10000
E = 320000
EL = 100000
D = 128
R_DEC = 2.0
T_DEC = 1.0

NC = 2
NS = 16
NW = NC * NS
CH = 128
KCH = 80
E2 = NW * KCH * CH
CHD = 128
KDC = 25
EL2 = NW * KDC * CHD
NBIN = N + 8
ROWS_PT = 624

_SC_MESH = plsc.VectorSubcoreMesh(core_axis_name="c", subcore_axis_name="s")


def _row_split(sid, fn):
    @pl.when(sid < NS - 1)
    def _():
        fn(sid * ROWS_PT, ROWS_PT)

    @pl.when(sid == NS - 1)
    def _():
        fn((NS - 1) * ROWS_PT, N - (NS - 1) * ROWS_PT)


def _hist_body(dst_hbm, zeros_hbm, deg0_hbm, deg1_hbm,
               acc, didx, ones_v, sem):
    del sem
    cid = lax.axis_index("c")
    sid = lax.axis_index("s")
    wid = cid * NS + sid

    ones = jnp.full((16,), 1.0, dtype=jnp.float32)
    for j in range(CH // 16):
        ones_v[pl.ds(j * 16, 16)] = ones
    pltpu.sync_copy(dst_hbm.at[wid], didx)

    @pl.when(sid == 0)
    def _():
        pltpu.sync_copy(zeros_hbm, acc)

    plsc.subcore_barrier()

    def chunk(k, carry):
        pltpu.sync_copy(ones_v, acc.at[didx.at[k]], add=True)
        return carry

    lax.fori_loop(0, KCH, chunk, 0)
    plsc.subcore_barrier()

    @pl.when(sid == 0)
    def _():
        @pl.when(cid == 0)
        def _():
            pltpu.sync_copy(acc, deg0_hbm)

        @pl.when(cid == 1)
        def _():
            pltpu.sync_copy(acc, deg1_hbm)


_hist_kernel = pl.kernel(
    _hist_body,
    out_type=(jax.ShapeDtypeStruct((NBIN,), jnp.float32),
              jax.ShapeDtypeStruct((NBIN,), jnp.float32)),
    mesh=_SC_MESH,
    scratch_types=[
        pltpu.VMEM_SHARED((NBIN,), jnp.float32),
        pltpu.VMEM((KCH, CH), jnp.int32),
        pltpu.VMEM((CH,), jnp.float32),
        pltpu.SemaphoreType.DMA,
    ],
)


def _agg_body(y_hbm, src_hbm, dst_hbm, zeros_hbm, outa_hbm, outb_hbm,
              acc, sidx, d0, d1, rows0, rows1, sem0, sem1, semd0, semd1):
    cid = lax.axis_index("c")
    sid = lax.axis_index("s")
    wid = cid * NS + sid

    pltpu.sync_copy(src_hbm.at[wid], sidx)

    def init(base, cnt):
        @pl.when(cid == 0)
        def _():
            pltpu.sync_copy(y_hbm.at[pl.ds(base, cnt)],
                            acc.at[pl.ds(base, cnt)])

        @pl.when(cid == 1)
        def _():
            pltpu.sync_copy(zeros_hbm.at[pl.ds(base, cnt)],
                            acc.at[pl.ds(base, cnt)])

    _row_split(sid, init)
    plsc.subcore_barrier()

    def gather(k, buf, sem):
        pltpu.async_copy(y_hbm.at[sidx.at[k]], buf, sem)

    def gwait(k, buf, sem):
        pltpu.make_async_copy(y_hbm.at[sidx.at[k]], buf, sem).wait()

    def dload(k, dbuf, semd):
        pltpu.async_copy(dst_hbm.at[wid, k], dbuf, semd)

    def dwait(k, dbuf, semd):
        pltpu.make_async_copy(dst_hbm.at[wid, k], dbuf, semd).wait()

    def scatter(buf, dbuf):
        pltpu.sync_copy(buf, acc.at[dbuf], add=True)

    gather(0, rows0, sem0)
    dload(0, d0, semd0)

    def pair(j, carry):
        k0 = 2 * j
        gwait(k0, rows0, sem0)
        gather(k0 + 1, rows1, sem1)
        dwait(k0, d0, semd0)
        dload(k0 + 1, d1, semd1)
        scatter(rows0, d0)
        gwait(k0 + 1, rows1, sem1)

        @pl.when(k0 + 2 < KCH)
        def _():
            gather(k0 + 2, rows0, sem0)

        dwait(k0 + 1, d1, semd1)

        @pl.when(k0 + 2 < KCH)
        def _():
            dload(k0 + 2, d0, semd0)

        scatter(rows1, d1)
        return carry

    lax.fori_loop(0, KCH // 2, pair, 0)
    plsc.subcore_barrier()

    def flush(base, cnt):
        @pl.when(cid == 0)
        def _():
            pltpu.sync_copy(acc.at[pl.ds(base, cnt)],
                            outa_hbm.at[pl.ds(base, cnt)])

        @pl.when(cid == 1)
        def _():
            pltpu.sync_copy(acc.at[pl.ds(base, cnt)],
                            outb_hbm.at[pl.ds(base, cnt)])

    _row_split(sid, flush)


_agg_kernel = pl.kernel(
    _agg_body,
    out_type=(jax.ShapeDtypeStruct((N, D), jnp.float32),
              jax.ShapeDtypeStruct((N, D), jnp.float32)),
    mesh=_SC_MESH,
    scratch_types=[
        pltpu.VMEM_SHARED((NBIN, D), jnp.float32),
        pltpu.VMEM((KCH, CH), jnp.int32),
        pltpu.VMEM((CH,), jnp.int32),
        pltpu.VMEM((CH,), jnp.int32),
        pltpu.VMEM((CH, D), jnp.float32),
        pltpu.VMEM((CH, D), jnp.float32),
        pltpu.SemaphoreType.DMA,
        pltpu.SemaphoreType.DMA,
        pltpu.SemaphoreType.DMA,
        pltpu.SemaphoreType.DMA,
    ],
)


def _dec_body(h_hbm, ein_hbm, eout_hbm, embi_hbm, embo_hbm,
              ia, ib, ra0, ra1, rb0, rb1, sa0, sa1, sb0, sb1):
    cid = lax.axis_index("c")
    sid = lax.axis_index("s")
    wid = cid * NS + sid

    pltpu.sync_copy(ein_hbm.at[wid], ia)
    pltpu.sync_copy(eout_hbm.at[wid], ib)

    def gather(k, bufa, bufb, sema, semb):
        pltpu.async_copy(h_hbm.at[ia.at[k]], bufa, sema)
        pltpu.async_copy(h_hbm.at[ib.at[k]], bufb, semb)

    def wait(k, bufa, bufb, sema, semb):
        pltpu.make_async_copy(h_hbm.at[ia.at[k]], bufa, sema).wait()
        pltpu.make_async_copy(h_hbm.at[ib.at[k]], bufb, semb).wait()

    def emit(bufa, bufb, k):
        base = (wid * KDC + k) * CHD
        pltpu.sync_copy(bufa, embi_hbm.at[pl.ds(base, CHD)])
        pltpu.sync_copy(bufb, embo_hbm.at[pl.ds(base, CHD)])

    gather(0, ra0, rb0, sa0, sb0)

    def pair(j, carry):
        k0 = 2 * j
        wait(k0, ra0, rb0, sa0, sb0)
        gather(k0 + 1, ra1, rb1, sa1, sb1)
        emit(ra0, rb0, k0)
        wait(k0 + 1, ra1, rb1, sa1, sb1)

        @pl.when(k0 + 2 < KDC)
        def _():
            gather(k0 + 2, ra0, rb0, sa0, sb0)

        emit(ra1, rb1, k0 + 1)
        return carry

    lax.fori_loop(0, KDC // 2, pair, 0)
    wait(KDC - 1, ra0, rb0, sa0, sb0)
    emit(ra0, rb0, KDC - 1)


_dec_kernel = pl.kernel(
    _dec_body,
    out_type=(jax.ShapeDtypeStruct((EL2, D), jnp.float32),
              jax.ShapeDtypeStruct((EL2, D), jnp.float32)),
    mesh=_SC_MESH,
    scratch_types=[
        pltpu.VMEM((KDC, CHD), jnp.int32),
        pltpu.VMEM((KDC, CHD), jnp.int32),
        pltpu.VMEM((CHD, D), jnp.float32),
        pltpu.VMEM((CHD, D), jnp.float32),
        pltpu.VMEM((CHD, D), jnp.float32),
        pltpu.VMEM((CHD, D), jnp.float32),
        pltpu.SemaphoreType.DMA,
        pltpu.SemaphoreType.DMA,
        pltpu.SemaphoreType.DMA,
        pltpu.SemaphoreType.DMA,
    ],
)


BLK = 1000


def _dinv(da_ref, db_ref):
    return lax.rsqrt(da_ref[...] + db_ref[...] + 1.0)


def _tc1_call(x, W1, dega, degb):
    def body(x_ref, w_ref, da_ref, db_ref, y_ref):
        xw = jnp.dot(x_ref[...], w_ref[...],
                     preferred_element_type=jnp.float32)
        y_ref[...] = xw * _dinv(da_ref, db_ref)

    return pl.pallas_call(
        body,
        grid=(N // BLK,),
        in_specs=[pl.BlockSpec((BLK, D), lambda i: (i, 0)),
                  pl.BlockSpec((D, D), lambda i: (0, 0)),
                  pl.BlockSpec((BLK, 1), lambda i: (i, 0)),
                  pl.BlockSpec((BLK, 1), lambda i: (i, 0))],
        out_specs=pl.BlockSpec((BLK, D), lambda i: (i, 0)),
        out_shape=jax.ShapeDtypeStruct((N, D), jnp.float32),
    )(x, W1, dega, degb)


def _tc2_call(a0, a1, dega, degb, b1, W2):
    def body(a0_ref, a1_ref, da_ref, db_ref, b_ref, w_ref, y_ref):
        dinv = _dinv(da_ref, db_ref)
        h = jnp.maximum(dinv * (a0_ref[...] + a1_ref[...]) + b_ref[...], 0.0)
        y_ref[...] = jnp.dot(h, w_ref[...],
                             preferred_element_type=jnp.float32) * dinv

    return pl.pallas_call(
        body,
        grid=(N // BLK,),
        in_specs=[pl.BlockSpec((BLK, D), lambda i: (i, 0)),
                  pl.BlockSpec((BLK, D), lambda i: (i, 0)),
                  pl.BlockSpec((BLK, 1), lambda i: (i, 0)),
                  pl.BlockSpec((BLK, 1), lambda i: (i, 0)),
                  pl.BlockSpec((D,), lambda i: (0,)),
                  pl.BlockSpec((D, D), lambda i: (0, 0))],
        out_specs=pl.BlockSpec((BLK, D), lambda i: (i, 0)),
        out_shape=jax.ShapeDtypeStruct((N, D), jnp.float32),
    )(a0, a1, dega, degb, b1, W2)


def _tc3_call(c0, c1, dega, degb, b2):
    def body(c0_ref, c1_ref, da_ref, db_ref, b_ref, h_ref):
        h_ref[...] = (_dinv(da_ref, db_ref) * (c0_ref[...] + c1_ref[...])
                      + b_ref[...])

    return pl.pallas_call(
        body,
        grid=(N // BLK,),
        in_specs=[pl.BlockSpec((BLK, D), lambda i: (i, 0)),
                  pl.BlockSpec((BLK, D), lambda i: (i, 0)),
                  pl.BlockSpec((BLK, 1), lambda i: (i, 0)),
                  pl.BlockSpec((BLK, 1), lambda i: (i, 0)),
                  pl.BlockSpec((D,), lambda i: (0,))],
        out_specs=pl.BlockSpec((BLK, D), lambda i: (i, 0)),
        out_shape=jax.ShapeDtypeStruct((N, D), jnp.float32),
    )(c0, c1, dega, degb, b2)


DBLK = 2048


def _tc4_call(embi, embo):
    def body(a_ref, b_ref, p_ref):
        d = a_ref[...] - b_ref[...]
        sq = jnp.sum(d * d, axis=1, keepdims=True)
        p_ref[...] = 1.0 / (jnp.exp((sq + 1e-12 - R_DEC) / T_DEC) + 1.0)

    return pl.pallas_call(
        body,
        grid=(EL2 // DBLK,),
        in_specs=[pl.BlockSpec((DBLK, D), lambda i: (i, 0)),
                  pl.BlockSpec((DBLK, D), lambda i: (i, 0))],
        out_specs=pl.BlockSpec((DBLK, 1), lambda i: (i, 0)),
        out_shape=jax.ShapeDtypeStruct((EL2, 1), jnp.float32),
    )(embi, embo)


def kernel(node_features, edge_index, edge_label_index, W1, b1, W2, b2):
    x = node_features.astype(jnp.float32)
    srcp = jnp.concatenate(
        [edge_index[0], jnp.zeros((E2 - E,), jnp.int32)]).reshape(NW, KCH, CH)
    dstp = jnp.concatenate(
        [edge_index[1], jnp.full((E2 - E,), N, jnp.int32)]).reshape(NW, KCH, CH)
    einp = jnp.concatenate(
        [edge_label_index[0],
         jnp.zeros((EL2 - EL,), jnp.int32)]).reshape(NW, KDC, CHD)
    eoutp = jnp.concatenate(
        [edge_label_index[1],
         jnp.zeros((EL2 - EL,), jnp.int32)]).reshape(NW, KDC, CHD)
    zeros_nd = jnp.zeros((N, D), jnp.float32)
    zeros_n = jnp.zeros((NBIN,), jnp.float32)

    deg0, deg1 = _hist_kernel(dstp, zeros_n)
    dega = deg0[:N].reshape(N, 1)
    degb = deg1[:N].reshape(N, 1)

    y1 = _tc1_call(x, W1, dega, degb)
    a0, a1 = _agg_kernel(y1, srcp, dstp, zeros_nd)
    y2 = _tc2_call(a0, a1, dega, degb, b1, W2)
    c0, c1 = _agg_kernel(y2, srcp, dstp, zeros_nd)
    h2 = _tc3_call(c0, c1, dega, degb, b2)

    embi, embo = _dec_kernel(h2, einp, eoutp)
    return _tc4_call(embi, embo)[:EL].reshape(EL)
